# Initial kernel scaffold; baseline (speedup 1.0000x reference)
import jax
import jax.numpy as jnp
from jax.experimental import pallas as pl


def kernel(batch_box_preds, batch_cls_preds):
    raise NotImplementedError("write your pallas kernel here")


# speedup vs baseline: 55.5951x; 55.5951x over previous
import jax
import jax.numpy as jnp
from jax.experimental import pallas as pl
from jax.experimental.pallas import tpu as pltpu

_POST = 500
_PRE = 4096
_IOU_T = 0.7
_SCORE_T = 0.1
_BLK = 128
_NBLK = _PRE // _BLK
_OUT_ROWS = 512
_OUT_COLS = 8


def _shift_right_lanes(x, d):
    return jnp.concatenate([jnp.zeros((x.shape[0], d), x.dtype), x[:, :-d]], axis=1)


def _nms_body(boxes_col_ref, boxes_row_ref, scores_row_ref, out_ref, sup_ref):
    x1r = boxes_row_ref[0:1, :]
    y1r = boxes_row_ref[1:2, :]
    x2r = boxes_row_ref[2:3, :]
    y2r = boxes_row_ref[3:4, :]
    area_row = jnp.clip(x2r - x1r, 0.0) * jnp.clip(y2r - y1r, 0.0)

    scores_row = scores_row_ref[0:1, :]
    lane = jax.lax.broadcasted_iota(jnp.int32, (1, _PRE), 1)
    keep = scores_row > _SCORE_T

    boxes_col = boxes_col_ref[:, :]

    for kb in range(_NBLK):
        cs = kb * _BLK
        ce = cs + _BLK
        bx1 = boxes_col[cs:ce, 0:1]
        by1 = boxes_col[cs:ce, 1:2]
        bx2 = boxes_col[cs:ce, 2:3]
        by2 = boxes_col[cs:ce, 3:4]
        barea = jnp.clip(bx2 - bx1, 0.0) * jnp.clip(by2 - by1, 0.0)

        xx1 = jnp.maximum(bx1, x1r)
        yy1 = jnp.maximum(by1, y1r)
        xx2 = jnp.minimum(bx2, x2r)
        yy2 = jnp.minimum(by2, y2r)
        inter = jnp.clip(xx2 - xx1, 0.0) * jnp.clip(yy2 - yy1, 0.0)
        union = barea + area_row - inter
        iou = inter / jnp.maximum(union, 1e-8)
        supf = jnp.where(iou > _IOU_T, 1.0, 0.0)

        sup_ref[:, :] = supf[:, cs:ce]
        blane = jax.lax.broadcasted_iota(jnp.int32, (1, _BLK), 1)
        keep_b0 = jnp.where(keep[:, cs:ce], 1.0, 0.0)

        def intra(j, kb_):
            row = sup_ref[pl.ds(j, 1), :]
            kj = jnp.sum(jnp.where(blane == j, kb_, 0.0), axis=1,
                         keepdims=True)
            sup = (row > 0.5) & (blane > j) & (kj > 0.5)
            return jnp.where(sup, 0.0, kb_)

        kbf = jax.lax.fori_loop(0, _BLK, intra, keep_b0)
        keep_b = kbf > 0.5

        cnt = jax.lax.dot_general(
            kbf, supf, (((1,), (0,)), ((), ())),
            preferred_element_type=jnp.float32)
        crossed = (cnt > 0.5) & (lane >= ce)
        pieces = []
        if cs > 0:
            pieces.append(keep[:, :cs])
        pieces.append(keep_b)
        if ce < _PRE:
            pieces.append(keep[:, ce:])
        keep = jnp.concatenate(pieces, axis=1) if len(pieces) > 1 else pieces[0]
        keep = keep & jnp.logical_not(crossed)

    kept_scores = jnp.where(keep, scores_row, -1.0)

    keepf = jnp.where(keep, 1.0, 0.0)
    csum = keepf
    d = 1
    while d < _PRE:
        csum = csum + _shift_right_lanes(csum, d)
        d *= 2
    total = csum[:, _PRE - 1:_PRE]
    lanef = lane.astype(jnp.float32)
    pos = jnp.where(keep, csum - 1.0, total + lanef - csum)

    rows = jax.lax.broadcasted_iota(jnp.int32, (_OUT_ROWS, 1), 0)
    onehot = jnp.where(rows == pos.astype(jnp.int32), 1.0, 0.0)

    out_boxes = jax.lax.dot_general(
        onehot, boxes_col, (((1,), (0,)), ((), ())),
        preferred_element_type=jnp.float32)
    out_scores = jnp.sum(onehot * kept_scores, axis=1, keepdims=True)
    pad = jnp.zeros((_OUT_ROWS, _OUT_COLS - 5), jnp.float32)
    out_ref[:, :] = jnp.concatenate([out_boxes, out_scores, pad], axis=1)


def _nms_core(boxes_col, boxes_row, scores_row):
    return pl.pallas_call(
        _nms_body,
        out_shape=jax.ShapeDtypeStruct((_OUT_ROWS, _OUT_COLS), jnp.float32),
        scratch_shapes=[pltpu.VMEM((_BLK, _BLK), jnp.float32)],
    )(boxes_col, boxes_row, scores_row)


def kernel(batch_box_preds, batch_cls_preds):
    boxes = batch_box_preds[0].T
    scores = jnp.max(batch_cls_preds[0], axis=0)
    masked = jnp.where(scores > _SCORE_T, scores, -1.0)
    top_scores, top_idx = jax.lax.top_k(masked, _PRE)
    top_boxes = jnp.take(boxes, top_idx, axis=0)
    out = _nms_core(top_boxes, top_boxes.T, top_scores[None, :])
    return out[:_POST, :6]
